# baseline (speedup 1.0000x reference)
import jax
import jax.numpy as jnp
from jax.experimental import pallas as pl


def _bcast_add_kernel(qpw_ref, q_ref, o_ref):
    s = q_ref[0] + qpw_ref[...]
    o_ref[...] = jnp.broadcast_to(s[None], o_ref.shape)


def kernel(x, query_pos_weight, queries):
    bs = x.shape[0]
    n_query, embed_dim = query_pos_weight.shape
    b_blk = 128
    grid = (bs // b_blk,)
    out = pl.pallas_call(
        _bcast_add_kernel,
        grid=grid,
        in_specs=[
            pl.BlockSpec((n_query, embed_dim), lambda i: (0, 0)),
            pl.BlockSpec((1, n_query, embed_dim), lambda i: (0, 0, 0)),
        ],
        out_specs=pl.BlockSpec((b_blk, n_query, embed_dim), lambda i: (i, 0, 0)),
        out_shape=jax.ShapeDtypeStruct((bs, n_query, embed_dim), queries.dtype),
    )(query_pos_weight, queries)
    return out


# speedup vs baseline: 1.0182x; 1.0182x over previous
import jax
import jax.numpy as jnp
from jax.experimental import pallas as pl
from jax.experimental.pallas import tpu as pltpu

_REP = 128


def _bcast_add_kernel(qpw_ref, q_ref, out_ref, rep_ref, sems):
    s = q_ref[0] + qpw_ref[...]
    rep_ref[...] = jnp.broadcast_to(s[None], rep_ref.shape)
    bs = out_ref.shape[0]
    n = bs // _REP
    copies = [
        pltpu.make_async_copy(
            rep_ref, out_ref.at[pl.ds(i * _REP, _REP)], sems.at[i]
        )
        for i in range(n)
    ]
    for c in copies:
        c.start()
    for c in copies:
        c.wait()


def kernel(x, query_pos_weight, queries):
    bs = x.shape[0]
    n_query, embed_dim = query_pos_weight.shape
    n = bs // _REP
    out = pl.pallas_call(
        _bcast_add_kernel,
        in_specs=[
            pl.BlockSpec(memory_space=pltpu.VMEM),
            pl.BlockSpec(memory_space=pltpu.VMEM),
        ],
        out_specs=pl.BlockSpec(memory_space=pl.ANY),
        out_shape=jax.ShapeDtypeStruct((bs, n_query, embed_dim), queries.dtype),
        scratch_shapes=[
            pltpu.VMEM((_REP, n_query, embed_dim), queries.dtype),
            pltpu.SemaphoreType.DMA((n,)),
        ],
    )(query_pos_weight, queries)
    return out
